# baseline (speedup 1.0000x reference)
import functools

import jax
import jax.numpy as jnp
from jax import lax
from jax.experimental import pallas as pl
from jax.experimental.pallas import tpu as pltpu
from jax.experimental.pallas import tpu_sc as plsc

N = 50000
D = 64
H = 32
E = 800000
NLAYERS = 3

NC, NS, L = 2, 16, 16

CHUNK = 128
GROUP = 2
CPT = 392
GPT = CPT // GROUP
EPAD = NS * CPT * CHUNK
NCHUNKS = EPAD // CHUNK

RPT = 3136
NPAD = NS * RPT
RCHUNK = 64
NRC = RPT // RCHUNK
DUMMY = N


def _rsqrt16(d):
    i = lax.bitcast_convert_type(d, jnp.int32)
    i = jnp.int32(0x5F3759DF) - lax.shift_right_arithmetic(i, 1)
    y = lax.bitcast_convert_type(i, jnp.float32)
    for _ in range(3):
        y = y * (1.5 - 0.5 * d * y * y)
    return y


def _body(embs, srcp, dstp, out, y0, y1, acc, deg,
          srcb0, srcb1, dstb0, dstb1, rows0, rows1,
          ab2, ob2, yb2, zb2, db, onesb, zb1, sem0, sem1):
    c = lax.axis_index("c")
    s = lax.axis_index("s")
    row_base = s * RPT
    chunk_base = s * CPT

    zero16 = jnp.zeros((L,), jnp.float32)
    one16 = jnp.ones((L,), jnp.float32)
    for r in range(RCHUNK):
        for h in range(H // L):
            zb2[r, pl.ds(h * L, L)] = zero16
    for v in range(RCHUNK // L):
        zb1[pl.ds(v * L, L)] = zero16
    for v in range(CHUNK // L):
        onesb[pl.ds(v * L, L)] = one16

    sc_p0 = jax.named_scope("p0_zero"); sc_p0.__enter__()
    def _zero_slices(i, carry):
        base = row_base + i * RCHUNK
        pltpu.sync_copy(zb1, deg.at[pl.ds(base, RCHUNK)])
        pltpu.sync_copy(zb2, acc.at[pl.ds(base, RCHUNK)])
        return carry
    lax.fori_loop(0, NRC, _zero_slices, 0)
    plsc.subcore_barrier()
    sc_p0.__exit__(None, None, None)
    sc_p1 = jax.named_scope("p1_deg"); sc_p1.__enter__()

    def _deg_group(g, carry):
        cb = chunk_base + g * 4
        pltpu.sync_copy(dstp.at[pl.ds(cb, 2)], dstb0)
        pltpu.sync_copy(dstp.at[pl.ds(cb + 2, 2)], dstb1)
        descs = [pltpu.async_copy(onesb, deg.at[dstb0.at[j]], sem0, add=True)
                 for j in range(2)]
        descs += [pltpu.async_copy(onesb, deg.at[dstb1.at[j]], sem0, add=True)
                  for j in range(2)]
        for dsc in descs:
            dsc.wait()
        return carry
    lax.fori_loop(0, CPT // 4, _deg_group, 0)
    plsc.subcore_barrier()
    sc_p1.__exit__(None, None, None)

    sc_p2 = jax.named_scope("p2_init"); sc_p2.__enter__()
    def _init_chunk(i, carry):
        base = row_base + i * RCHUNK
        pltpu.sync_copy(deg.at[pl.ds(base, RCHUNK)], db)
        pltpu.sync_copy(embs.at[c].at[pl.ds(base, RCHUNK)], ab2)

        def _rows(v, carry2):
            d = jnp.maximum(db[pl.ds(v * L, L)], 1.0)
            sdvec = _rsqrt16(d)
            for j in range(L):
                r = v * L + j
                sd = sdvec[j]
                for h in range(H // L):
                    e = ab2[r, pl.ds(h * L, L)]
                    yb2[r, pl.ds(h * L, L)] = e * sd
            return carry2
        lax.fori_loop(0, RCHUNK // L, _rows, 0)
        pltpu.sync_copy(ab2, out.at[c].at[pl.ds(base, RCHUNK)])
        pltpu.sync_copy(yb2, y0.at[c].at[pl.ds(base, RCHUNK)])
        return carry
    lax.fori_loop(0, NRC, _init_chunk, 0)
    plsc.subcore_barrier()
    sc_p2.__exit__(None, None, None)

    ybufs = [y0, y1]
    for layer in range(NLAYERS):
        ycur = ybufs[layer % 2]
        ynext = ybufs[(layer + 1) % 2]

        def _fire(g, srcb, dstb, rows, sem):
            cb = chunk_base + g * GROUP
            pltpu.sync_copy(srcp.at[pl.ds(cb, GROUP)], srcb)
            pltpu.sync_copy(dstp.at[pl.ds(cb, GROUP)], dstb)
            for j in range(GROUP):
                pltpu.async_copy(ycur.at[c].at[srcb.at[j]],
                                 rows.at[pl.ds(j * CHUNK, CHUNK)], sem)

        def _drain_scatter(srcb, dstb, rows, sem):
            for j in range(GROUP):
                pltpu.make_async_copy(
                    ycur.at[c].at[srcb.at[j]],
                    rows.at[pl.ds(j * CHUNK, CHUNK)], sem).wait()
            for j in range(GROUP):
                pltpu.sync_copy(rows.at[pl.ds(j * CHUNK, CHUNK)],
                                acc.at[dstb.at[j]], add=True)

        sc_agg = jax.named_scope(f"agg{layer}"); sc_agg.__enter__()
        _fire(0, srcb0, dstb0, rows0, sem0)

        def _agg_pair(i, carry):
            _fire(2 * i + 1, srcb1, dstb1, rows1, sem1)
            _drain_scatter(srcb0, dstb0, rows0, sem0)

            @pl.when(i < GPT // 2 - 1)
            def _():
                _fire(2 * i + 2, srcb0, dstb0, rows0, sem0)
            _drain_scatter(srcb1, dstb1, rows1, sem1)
            return carry
        lax.fori_loop(0, GPT // 2, _agg_pair, 0)
        plsc.subcore_barrier()
        sc_agg.__exit__(None, None, None)
        sc_post = jax.named_scope(f"post{layer}"); sc_post.__enter__()

        last = layer == NLAYERS - 1

        def _post_chunk(i, carry):
            base = row_base + i * RCHUNK
            pltpu.sync_copy(deg.at[pl.ds(base, RCHUNK)], db)
            pltpu.sync_copy(acc.at[pl.ds(base, RCHUNK)], ab2)
            pltpu.sync_copy(out.at[c].at[pl.ds(base, RCHUNK)], ob2)

            def _rows(v, carry2):
                d = jnp.maximum(db[pl.ds(v * L, L)], 1.0)
                sdvec = _rsqrt16(d)
                for j in range(L):
                    r = v * L + j
                    sd = sdvec[j]
                    for h in range(H // L):
                        sl = pl.ds(h * L, L)
                        t = ab2[r, sl] * sd
                        o = ob2[r, sl] + t
                        if last:
                            ob2[r, sl] = o * 0.25
                        else:
                            ob2[r, sl] = o
                            yb2[r, sl] = t * sd
                return carry2
            lax.fori_loop(0, RCHUNK // L, _rows, 0)
            pltpu.sync_copy(zb2, acc.at[pl.ds(base, RCHUNK)])
            pltpu.sync_copy(ob2, out.at[c].at[pl.ds(base, RCHUNK)])
            if not last:
                pltpu.sync_copy(yb2, ynext.at[c].at[pl.ds(base, RCHUNK)])
            return carry
        lax.fori_loop(0, NRC, _post_chunk, 0)
        plsc.subcore_barrier()
        sc_post.__exit__(None, None, None)


@jax.jit
def _run(embs, srcp, dstp):
    mesh = plsc.VectorSubcoreMesh(core_axis_name="c", subcore_axis_name="s")
    f = pl.kernel(
        _body,
        out_type=[
            jax.ShapeDtypeStruct((NC, NPAD, H), jnp.float32),
            jax.ShapeDtypeStruct((NC, NPAD, H), jnp.float32),
            jax.ShapeDtypeStruct((NC, NPAD, H), jnp.float32),
        ],
        mesh=mesh,
        compiler_params=pltpu.CompilerParams(use_tc_tiling_on_sc=False),
        scratch_types=[
            pltpu.VMEM_SHARED((NPAD, H), jnp.float32),
            pltpu.VMEM_SHARED((NPAD,), jnp.float32),
            pltpu.VMEM((GROUP, CHUNK), jnp.int32),
            pltpu.VMEM((GROUP, CHUNK), jnp.int32),
            pltpu.VMEM((GROUP, CHUNK), jnp.int32),
            pltpu.VMEM((GROUP, CHUNK), jnp.int32),
            pltpu.VMEM((GROUP * CHUNK, H), jnp.float32),
            pltpu.VMEM((GROUP * CHUNK, H), jnp.float32),
            pltpu.VMEM((RCHUNK, H), jnp.float32),
            pltpu.VMEM((RCHUNK, H), jnp.float32),
            pltpu.VMEM((RCHUNK, H), jnp.float32),
            pltpu.VMEM((RCHUNK, H), jnp.float32),
            pltpu.VMEM((RCHUNK,), jnp.float32),
            pltpu.VMEM((CHUNK,), jnp.float32),
            pltpu.VMEM((RCHUNK,), jnp.float32),
            pltpu.SemaphoreType.DMA,
            pltpu.SemaphoreType.DMA,
        ],
    )
    return f(embs, srcp, dstp)


def kernel(emb, edge_index):
    src = edge_index[0]
    dst = edge_index[1]
    pad = jnp.full((EPAD - E,), DUMMY, dtype=jnp.int32)
    srcp = jnp.concatenate([src, pad]).reshape(NCHUNKS, CHUNK)
    dstp = jnp.concatenate([dst, pad]).reshape(NCHUNKS, CHUNK)
    embp = jnp.pad(emb, ((0, NPAD - N), (0, 0)))
    embs = jnp.stack([embp[:, :H], embp[:, H:]])
    out, _, _ = _run(embs, srcp, dstp)
    return jnp.concatenate([out[0, :N], out[1, :N]], axis=1)


# speedup vs baseline: 1.0995x; 1.0995x over previous
import jax
import jax.numpy as jnp
from jax import lax
from jax.experimental import pallas as pl
from jax.experimental.pallas import tpu as pltpu
from jax.experimental.pallas import tpu_sc as plsc

N = 50000
D = 64
H = 32
E = 800000
NLAYERS = 3

NC, NS, L = 2, 16, 16

CHUNK = 128
IB = 7
CPT = 392
NB = CPT // IB
NBP = NB // 2
EPAD = NS * CPT * CHUNK
NCHUNKS = EPAD // CHUNK

RPT = 3136
NPAD = NS * RPT
RCHUNK = 64
NRC = RPT // RCHUNK
DUMMY = N


def _rsqrt16(d):
    i = lax.bitcast_convert_type(d, jnp.int32)
    i = jnp.int32(0x5F3759DF) - lax.shift_right_arithmetic(i, 1)
    y = lax.bitcast_convert_type(i, jnp.float32)
    for _ in range(3):
        y = y * (1.5 - 0.5 * d * y * y)
    return y


def _body(embp, srcp, dstp, out, y0, y1, sdb, acc, deg,
          srcbA, dstbA, srcbB, dstbB, rows0, rows1,
          ab2, ob2, yb2, sb2, zb2, db, onesb, zb1, sem0, sem1):
    c = lax.axis_index("c")
    s = lax.axis_index("s")
    row_base = s * RPT
    chunk_base = s * CPT
    col = c * H

    zero16 = jnp.zeros((L,), jnp.float32)
    one16 = jnp.ones((L,), jnp.float32)
    for r in range(RCHUNK):
        for h in range(H // L):
            zb2[r, pl.ds(h * L, L)] = zero16
    for v in range(RCHUNK // L):
        zb1[pl.ds(v * L, L)] = zero16
    for v in range(CHUNK // L):
        onesb[pl.ds(v * L, L)] = one16

    def _zero_slices(i, carry):
        base = row_base + i * RCHUNK
        pltpu.sync_copy(zb1, deg.at[pl.ds(base, RCHUNK)])
        pltpu.sync_copy(zb2, acc.at[pl.ds(base, RCHUNK)])
        return carry
    lax.fori_loop(0, NRC, _zero_slices, 0)
    plsc.subcore_barrier()

    def _deg_batch(b, carry):
        cb = chunk_base + b * IB
        pltpu.sync_copy(dstp.at[pl.ds(cb, IB)], dstbA)
        descs = [pltpu.async_copy(onesb, deg.at[dstbA.at[j]], sem0, add=True)
                 for j in range(IB)]
        for dsc in descs:
            dsc.wait()
        return carry
    lax.fori_loop(0, NB, _deg_batch, 0)
    plsc.subcore_barrier()

    def _init_chunk(i, carry):
        base = row_base + i * RCHUNK
        pltpu.sync_copy(deg.at[pl.ds(base, RCHUNK)], db)
        pltpu.sync_copy(embp.at[pl.ds(base, RCHUNK), pl.ds(col, H)], ab2)

        def _rows(v, carry2):
            d = jnp.maximum(db[pl.ds(v * L, L)], 1.0)
            sdvec = _rsqrt16(d)
            for j in range(L):
                r = v * L + j
                sd = sdvec[j]
                for h in range(H // L):
                    sl = pl.ds(h * L, L)
                    sb2[r, sl] = one16 * sd
                    yb2[r, sl] = ab2[r, sl] * sd
            return carry2
        lax.fori_loop(0, RCHUNK // L, _rows, 0)
        pltpu.sync_copy(sb2, sdb.at[c].at[pl.ds(base, RCHUNK)])
        pltpu.sync_copy(ab2, out.at[pl.ds(base, RCHUNK), pl.ds(col, H)])
        pltpu.sync_copy(yb2, y0.at[c].at[pl.ds(base, RCHUNK)])
        return carry
    lax.fori_loop(0, NRC, _init_chunk, 0)
    plsc.subcore_barrier()

    ybufs = [y0, y1]
    for layer in range(NLAYERS):
        ycur = ybufs[layer % 2]
        ynext = ybufs[(layer + 1) % 2]

        rowbufs = [rows0, rows1]
        sems = [sem0, sem1]

        def _fire(sbuf, j, p):
            pltpu.async_copy(ycur.at[c].at[sbuf.at[j]], rowbufs[p], sems[p])

        def _drain_scatter(sbuf, dbuf, j, p):
            pltpu.make_async_copy(ycur.at[c].at[sbuf.at[j]],
                                  rowbufs[p], sems[p]).wait()
            pltpu.sync_copy(rowbufs[p], acc.at[dbuf.at[j]], add=True)

        def _load(b, sbuf, dbuf):
            cb = chunk_base + b * IB
            pltpu.sync_copy(srcp.at[pl.ds(cb, IB)], sbuf)
            pltpu.sync_copy(dstp.at[pl.ds(cb, IB)], dbuf)

        _load(0, srcbA, dstbA)
        _fire(srcbA, 0, 0)

        def _half_batch(i, b, sbuf, dbuf, nbuf_s, nbuf_d, p0, is_last):
            for j in range(1, IB):
                _fire(sbuf, j, (p0 + j) % 2)
                _drain_scatter(sbuf, dbuf, j - 1, (p0 + j - 1) % 2)
            if is_last:
                @pl.when(i < NBP - 1)
                def _():
                    _load(b + 1, nbuf_s, nbuf_d)
                    _fire(nbuf_s, 0, (p0 + IB) % 2)
            else:
                _load(b + 1, nbuf_s, nbuf_d)
                _fire(nbuf_s, 0, (p0 + IB) % 2)
            _drain_scatter(sbuf, dbuf, IB - 1, (p0 + IB - 1) % 2)

        def _agg_pair(i, carry):
            _half_batch(i, 2 * i, srcbA, dstbA, srcbB, dstbB, 0, False)
            _half_batch(i, 2 * i + 1, srcbB, dstbB, srcbA, dstbA, 1, True)
            return carry
        lax.fori_loop(0, NBP, _agg_pair, 0)
        plsc.subcore_barrier()

        last = layer == NLAYERS - 1

        def _post_chunk(i, carry):
            base = row_base + i * RCHUNK
            pltpu.sync_copy(sdb.at[c].at[pl.ds(base, RCHUNK)], sb2)
            pltpu.sync_copy(acc.at[pl.ds(base, RCHUNK)], ab2)
            pltpu.sync_copy(out.at[pl.ds(base, RCHUNK), pl.ds(col, H)], ob2)

            def _rows(r0, carry2):
                for k in range(4):
                    r = r0 * 4 + k
                    for h in range(H // L):
                        sl = pl.ds(h * L, L)
                        sd = sb2[r, sl]
                        t = ab2[r, sl] * sd
                        o = ob2[r, sl] + t
                        if last:
                            ob2[r, sl] = o * 0.25
                        else:
                            ob2[r, sl] = o
                            yb2[r, sl] = t * sd
                return carry2
            lax.fori_loop(0, RCHUNK // 4, _rows, 0)
            pltpu.sync_copy(zb2, acc.at[pl.ds(base, RCHUNK)])
            pltpu.sync_copy(ob2, out.at[pl.ds(base, RCHUNK), pl.ds(col, H)])
            if not last:
                pltpu.sync_copy(yb2, ynext.at[c].at[pl.ds(base, RCHUNK)])
            return carry
        lax.fori_loop(0, NRC, _post_chunk, 0)
        plsc.subcore_barrier()


@jax.jit
def _run(embp, srcp, dstp):
    mesh = plsc.VectorSubcoreMesh(core_axis_name="c", subcore_axis_name="s")
    f = pl.kernel(
        _body,
        out_type=[
            jax.ShapeDtypeStruct((NPAD, D), jnp.float32),
            jax.ShapeDtypeStruct((NC, NPAD, H), jnp.float32),
            jax.ShapeDtypeStruct((NC, NPAD, H), jnp.float32),
            jax.ShapeDtypeStruct((NC, NPAD, H), jnp.float32),
        ],
        mesh=mesh,
        compiler_params=pltpu.CompilerParams(use_tc_tiling_on_sc=False),
        scratch_types=[
            pltpu.VMEM_SHARED((NPAD, H), jnp.float32),
            pltpu.VMEM_SHARED((NPAD,), jnp.float32),
            pltpu.VMEM((IB, CHUNK), jnp.int32),
            pltpu.VMEM((IB, CHUNK), jnp.int32),
            pltpu.VMEM((IB, CHUNK), jnp.int32),
            pltpu.VMEM((IB, CHUNK), jnp.int32),
            pltpu.VMEM((CHUNK, H), jnp.float32),
            pltpu.VMEM((CHUNK, H), jnp.float32),
            pltpu.VMEM((RCHUNK, H), jnp.float32),
            pltpu.VMEM((RCHUNK, H), jnp.float32),
            pltpu.VMEM((RCHUNK, H), jnp.float32),
            pltpu.VMEM((RCHUNK, H), jnp.float32),
            pltpu.VMEM((RCHUNK, H), jnp.float32),
            pltpu.VMEM((RCHUNK,), jnp.float32),
            pltpu.VMEM((CHUNK,), jnp.float32),
            pltpu.VMEM((RCHUNK,), jnp.float32),
            pltpu.SemaphoreType.DMA,
            pltpu.SemaphoreType.DMA,
        ],
    )
    return f(embp, srcp, dstp)


def kernel(emb, edge_index):
    src = edge_index[0]
    dst = edge_index[1]
    pad = jnp.full((EPAD - E,), DUMMY, dtype=jnp.int32)
    srcp = jnp.concatenate([src, pad]).reshape(NCHUNKS, CHUNK)
    dstp = jnp.concatenate([dst, pad]).reshape(NCHUNKS, CHUNK)
    embp = jnp.pad(emb, ((0, NPAD - N), (0, 0)))
    out, _, _, _ = _run(embp, srcp, dstp)
    return out[:N]


# speedup vs baseline: 1.1140x; 1.0132x over previous
import jax
import jax.numpy as jnp
from jax import lax
from jax.experimental import pallas as pl
from jax.experimental.pallas import tpu as pltpu
from jax.experimental.pallas import tpu_sc as plsc

N = 50000
D = 64
H = 32
E = 800000
NLAYERS = 3

NC, NS, L = 2, 16, 16

CHUNK = 128
IB = 7
CPT = 392
NB = CPT // IB
NBP = NB // 2
EPAD = NS * CPT * CHUNK
NCHUNKS = EPAD // CHUNK

RPT = 3136
NPAD = NS * RPT
RCHUNK = 64
NRC = RPT // RCHUNK
DUMMY = N


def _rsqrt16(d):
    i = lax.bitcast_convert_type(d, jnp.int32)
    i = jnp.int32(0x5F3759DF) - lax.shift_right_arithmetic(i, 1)
    y = lax.bitcast_convert_type(i, jnp.float32)
    for _ in range(3):
        y = y * (1.5 - 0.5 * d * y * y)
    return y


def _body(embp, srcp, dstp, out, y0, y1, sdb, acc, deg,
          srcbA, dstbA, srcbB, dstbB, rows0, rows1,
          ab2, ob2, yb2, sb2, zb2, db, onesb, zb1, dumidx,
          sem0, sem1, ssem0, ssem1):
    c = lax.axis_index("c")
    s = lax.axis_index("s")
    row_base = s * RPT
    chunk_base = s * CPT
    col = c * H

    zero16 = jnp.zeros((L,), jnp.float32)
    one16 = jnp.ones((L,), jnp.float32)
    for r in range(RCHUNK):
        for h in range(H // L):
            zb2[r, pl.ds(h * L, L)] = zero16
    for v in range(RCHUNK // L):
        zb1[pl.ds(v * L, L)] = zero16
    for v in range(CHUNK // L):
        onesb[pl.ds(v * L, L)] = one16
        dumidx[pl.ds(v * L, L)] = jnp.full((L,), DUMMY, jnp.int32)

    def _zero_slices(i, carry):
        base = row_base + i * RCHUNK
        pltpu.sync_copy(zb1, deg.at[pl.ds(base, RCHUNK)])
        pltpu.sync_copy(zb2, acc.at[pl.ds(base, RCHUNK)])
        return carry
    lax.fori_loop(0, NRC, _zero_slices, 0)
    plsc.subcore_barrier()

    def _deg_batch(b, carry):
        cb = chunk_base + b * IB
        pltpu.sync_copy(dstp.at[pl.ds(cb, IB)], dstbA)
        descs = [pltpu.async_copy(onesb, deg.at[dstbA.at[j]], sem0, add=True)
                 for j in range(IB)]
        for dsc in descs:
            dsc.wait()
        return carry
    lax.fori_loop(0, NB, _deg_batch, 0)
    plsc.subcore_barrier()

    def _init_chunk(i, carry):
        base = row_base + i * RCHUNK
        pltpu.sync_copy(deg.at[pl.ds(base, RCHUNK)], db)
        pltpu.sync_copy(embp.at[pl.ds(base, RCHUNK), pl.ds(col, H)], ab2)

        def _rows(v, carry2):
            d = jnp.maximum(db[pl.ds(v * L, L)], 1.0)
            sdvec = _rsqrt16(d)
            for j in range(L):
                r = v * L + j
                sd = sdvec[j]
                for h in range(H // L):
                    sl = pl.ds(h * L, L)
                    sb2[r, sl] = one16 * sd
                    yb2[r, sl] = ab2[r, sl] * sd
            return carry2
        lax.fori_loop(0, RCHUNK // L, _rows, 0)
        pltpu.sync_copy(sb2, sdb.at[c].at[pl.ds(base, RCHUNK)])
        pltpu.sync_copy(ab2, out.at[pl.ds(base, RCHUNK), pl.ds(col, H)])
        pltpu.sync_copy(yb2, y0.at[c].at[pl.ds(base, RCHUNK)])
        return carry
    lax.fori_loop(0, NRC, _init_chunk, 0)
    plsc.subcore_barrier()

    ybufs = [y0, y1]
    for layer in range(NLAYERS):
        ycur = ybufs[layer % 2]
        ynext = ybufs[(layer + 1) % 2]

        rowbufs = [rows0, rows1]
        sems = [sem0, sem1]
        ssems = [ssem0, ssem1]

        def _wait_scatter(p):
            pltpu.make_async_copy(rowbufs[p], acc.at[dumidx],
                                  ssems[p]).wait()

        def _fire(sbuf, j, p):
            _wait_scatter(p)
            pltpu.async_copy(ycur.at[c].at[sbuf.at[j]], rowbufs[p], sems[p])

        def _scat(sbuf, dbuf, j, p):
            pltpu.make_async_copy(ycur.at[c].at[sbuf.at[j]],
                                  rowbufs[p], sems[p]).wait()
            pltpu.async_copy(rowbufs[p], acc.at[dbuf.at[j]], ssems[p],
                             add=True)

        def _load(b, sbuf, dbuf):
            cb = chunk_base + b * IB
            pltpu.sync_copy(srcp.at[pl.ds(cb, IB)], sbuf)
            pltpu.sync_copy(dstp.at[pl.ds(cb, IB)], dbuf)

        for p in range(2):
            pltpu.async_copy(rowbufs[p], acc.at[dumidx], ssems[p], add=True)
        _load(0, srcbA, dstbA)
        _fire(srcbA, 0, 0)

        def _half_batch(i, b, sbuf, dbuf, nbuf_s, nbuf_d, p0, is_last):
            for j in range(1, IB):
                _fire(sbuf, j, (p0 + j) % 2)
                _scat(sbuf, dbuf, j - 1, (p0 + j - 1) % 2)
            if is_last:
                @pl.when(i < NBP - 1)
                def _():
                    _load(b + 1, nbuf_s, nbuf_d)
                    _fire(nbuf_s, 0, (p0 + IB) % 2)
            else:
                _load(b + 1, nbuf_s, nbuf_d)
                _fire(nbuf_s, 0, (p0 + IB) % 2)
            _scat(sbuf, dbuf, IB - 1, (p0 + IB - 1) % 2)

        def _agg_pair(i, carry):
            _half_batch(i, 2 * i, srcbA, dstbA, srcbB, dstbB, 0, False)
            _half_batch(i, 2 * i + 1, srcbB, dstbB, srcbA, dstbA, 1, True)
            return carry
        lax.fori_loop(0, NBP, _agg_pair, 0)
        _wait_scatter(0)
        _wait_scatter(1)
        plsc.subcore_barrier()

        last = layer == NLAYERS - 1

        def _post_chunk(i, carry):
            base = row_base + i * RCHUNK
            pltpu.sync_copy(sdb.at[c].at[pl.ds(base, RCHUNK)], sb2)
            pltpu.sync_copy(acc.at[pl.ds(base, RCHUNK)], ab2)
            pltpu.sync_copy(out.at[pl.ds(base, RCHUNK), pl.ds(col, H)], ob2)

            def _rows(r0, carry2):
                for k in range(4):
                    r = r0 * 4 + k
                    for h in range(H // L):
                        sl = pl.ds(h * L, L)
                        sd = sb2[r, sl]
                        t = ab2[r, sl] * sd
                        o = ob2[r, sl] + t
                        if last:
                            ob2[r, sl] = o * 0.25
                        else:
                            ob2[r, sl] = o
                            yb2[r, sl] = t * sd
                return carry2
            lax.fori_loop(0, RCHUNK // 4, _rows, 0)
            pltpu.sync_copy(zb2, acc.at[pl.ds(base, RCHUNK)])
            pltpu.sync_copy(ob2, out.at[pl.ds(base, RCHUNK), pl.ds(col, H)])
            if not last:
                pltpu.sync_copy(yb2, ynext.at[c].at[pl.ds(base, RCHUNK)])
            return carry
        lax.fori_loop(0, NRC, _post_chunk, 0)
        plsc.subcore_barrier()


@jax.jit
def _run(embp, srcp, dstp):
    mesh = plsc.VectorSubcoreMesh(core_axis_name="c", subcore_axis_name="s")
    f = pl.kernel(
        _body,
        out_type=[
            jax.ShapeDtypeStruct((NPAD, D), jnp.float32),
            jax.ShapeDtypeStruct((NC, NPAD, H), jnp.float32),
            jax.ShapeDtypeStruct((NC, NPAD, H), jnp.float32),
            jax.ShapeDtypeStruct((NC, NPAD, H), jnp.float32),
        ],
        mesh=mesh,
        compiler_params=pltpu.CompilerParams(use_tc_tiling_on_sc=False),
        scratch_types=[
            pltpu.VMEM_SHARED((NPAD, H), jnp.float32),
            pltpu.VMEM_SHARED((NPAD,), jnp.float32),
            pltpu.VMEM((IB, CHUNK), jnp.int32),
            pltpu.VMEM((IB, CHUNK), jnp.int32),
            pltpu.VMEM((IB, CHUNK), jnp.int32),
            pltpu.VMEM((IB, CHUNK), jnp.int32),
            pltpu.VMEM((CHUNK, H), jnp.float32),
            pltpu.VMEM((CHUNK, H), jnp.float32),
            pltpu.VMEM((RCHUNK, H), jnp.float32),
            pltpu.VMEM((RCHUNK, H), jnp.float32),
            pltpu.VMEM((RCHUNK, H), jnp.float32),
            pltpu.VMEM((RCHUNK, H), jnp.float32),
            pltpu.VMEM((RCHUNK, H), jnp.float32),
            pltpu.VMEM((RCHUNK,), jnp.float32),
            pltpu.VMEM((CHUNK,), jnp.float32),
            pltpu.VMEM((RCHUNK,), jnp.float32),
            pltpu.VMEM((CHUNK,), jnp.int32),
            pltpu.SemaphoreType.DMA,
            pltpu.SemaphoreType.DMA,
            pltpu.SemaphoreType.DMA,
            pltpu.SemaphoreType.DMA,
        ],
    )
    return f(embp, srcp, dstp)


def kernel(emb, edge_index):
    src = edge_index[0]
    dst = edge_index[1]
    pad = jnp.full((EPAD - E,), DUMMY, dtype=jnp.int32)
    srcp = jnp.concatenate([src, pad]).reshape(NCHUNKS, CHUNK)
    dstp = jnp.concatenate([dst, pad]).reshape(NCHUNKS, CHUNK)
    embp = jnp.pad(emb, ((0, NPAD - N), (0, 0)))
    out, _, _, _ = _run(embp, srcp, dstp)
    return out[:N]


# speedup vs baseline: 1.1277x; 1.0123x over previous
import jax
import jax.numpy as jnp
from jax import lax
from jax.experimental import pallas as pl
from jax.experimental.pallas import tpu as pltpu
from jax.experimental.pallas import tpu_sc as plsc

N = 50000
D = 64
H = 32
E = 800000
NLAYERS = 3

NC, NS, L = 2, 16, 16

CHUNK = 128
IB = 7
CPT = 392
NB = CPT // IB
NBP = NB // 2
EPAD = NS * CPT * CHUNK
NCHUNKS = EPAD // CHUNK

RPT = 3136
NPAD = NS * RPT
RCHUNK = 64
NRC = RPT // RCHUNK
DUMMY = N


def _rsqrt16(d):
    i = lax.bitcast_convert_type(d, jnp.int32)
    i = jnp.int32(0x5F3759DF) - lax.shift_right_arithmetic(i, 1)
    y = lax.bitcast_convert_type(i, jnp.float32)
    for _ in range(3):
        y = y * (1.5 - 0.5 * d * y * y)
    return y


def _body(embp, srcp, dstp, out, y0, y1, sdb, acc, deg,
          srcbA, dstbA, srcbB, dstbB, rows0, rows1,
          ab2, ob2, yb2, sb2, zb2, db, onesb, zb1, dumidx,
          sem0, sem1, ssem0, ssem1):
    c = lax.axis_index("c")
    s = lax.axis_index("s")
    row_base = s * RPT
    chunk_base = s * CPT
    col = c * H

    zero16 = jnp.zeros((L,), jnp.float32)
    one16 = jnp.ones((L,), jnp.float32)
    for r in range(RCHUNK):
        for h in range(H // L):
            zb2[r, pl.ds(h * L, L)] = zero16
    for v in range(RCHUNK // L):
        zb1[pl.ds(v * L, L)] = zero16
    for v in range(CHUNK // L):
        onesb[pl.ds(v * L, L)] = one16
        dumidx[pl.ds(v * L, L)] = jnp.full((L,), DUMMY, jnp.int32)

    def _zero_slices(i, carry):
        base = row_base + i * RCHUNK
        pltpu.sync_copy(zb1, deg.at[pl.ds(base, RCHUNK)])
        pltpu.sync_copy(zb2, acc.at[pl.ds(base, RCHUNK)])
        return carry
    lax.fori_loop(0, NRC, _zero_slices, 0)
    plsc.subcore_barrier()

    pltpu.sync_copy(dstp.at[pl.ds(chunk_base, IB)], dstbA)

    def _deg_pair(i, carry):
        b = 2 * i
        descsA = [pltpu.async_copy(onesb, deg.at[dstbA.at[j]], sem0, add=True)
                  for j in range(IB)]
        pltpu.sync_copy(dstp.at[pl.ds(chunk_base + (b + 1) * IB, IB)], dstbB)
        descsB = [pltpu.async_copy(onesb, deg.at[dstbB.at[j]], sem1, add=True)
                  for j in range(IB)]
        for dsc in descsA:
            dsc.wait()

        @pl.when(i < NB // 2 - 1)
        def _():
            pltpu.sync_copy(dstp.at[pl.ds(chunk_base + (b + 2) * IB, IB)],
                            dstbA)
        for dsc in descsB:
            dsc.wait()
        return carry
    lax.fori_loop(0, NB // 2, _deg_pair, 0)
    plsc.subcore_barrier()

    def _init_chunk(i, carry):
        base = row_base + i * RCHUNK
        pltpu.sync_copy(deg.at[pl.ds(base, RCHUNK)], db)
        pltpu.sync_copy(embp.at[pl.ds(base, RCHUNK), pl.ds(col, H)], ab2)

        def _rows(v, carry2):
            d = jnp.maximum(db[pl.ds(v * L, L)], 1.0)
            sdvec = _rsqrt16(d)
            for j in range(L):
                r = v * L + j
                sd = sdvec[j]
                for h in range(H // L):
                    sl = pl.ds(h * L, L)
                    sb2[r, sl] = one16 * sd
                    yb2[r, sl] = ab2[r, sl] * sd
            return carry2
        lax.fori_loop(0, RCHUNK // L, _rows, 0)
        pltpu.sync_copy(sb2, sdb.at[c].at[pl.ds(base, RCHUNK)])
        pltpu.sync_copy(ab2, out.at[pl.ds(base, RCHUNK), pl.ds(col, H)])
        pltpu.sync_copy(yb2, y0.at[c].at[pl.ds(base, RCHUNK)])
        return carry
    lax.fori_loop(0, NRC, _init_chunk, 0)
    plsc.subcore_barrier()

    ybufs = [y0, y1]
    for layer in range(NLAYERS):
        ycur = ybufs[layer % 2]
        ynext = ybufs[(layer + 1) % 2]

        rowbufs = [rows0, rows1]
        sems = [sem0, sem1]
        ssems = [ssem0, ssem1]

        def _wait_scatter(p):
            pltpu.make_async_copy(rowbufs[p], acc.at[dumidx],
                                  ssems[p]).wait()

        def _fire(sbuf, j, p):
            _wait_scatter(p)
            pltpu.async_copy(ycur.at[c].at[sbuf.at[j]], rowbufs[p], sems[p])

        def _scat(sbuf, dbuf, j, p):
            pltpu.make_async_copy(ycur.at[c].at[sbuf.at[j]],
                                  rowbufs[p], sems[p]).wait()
            pltpu.async_copy(rowbufs[p], acc.at[dbuf.at[j]], ssems[p],
                             add=True)

        def _load(b, sbuf, dbuf):
            cb = chunk_base + b * IB
            pltpu.sync_copy(srcp.at[pl.ds(cb, IB)], sbuf)
            pltpu.sync_copy(dstp.at[pl.ds(cb, IB)], dbuf)

        for p in range(2):
            pltpu.async_copy(rowbufs[p], acc.at[dumidx], ssems[p], add=True)
        _load(0, srcbA, dstbA)
        _fire(srcbA, 0, 0)

        def _half_batch(i, b, sbuf, dbuf, nbuf_s, nbuf_d, p0, is_last):
            for j in range(1, IB):
                _fire(sbuf, j, (p0 + j) % 2)
                _scat(sbuf, dbuf, j - 1, (p0 + j - 1) % 2)
            if is_last:
                @pl.when(i < NBP - 1)
                def _():
                    _load(b + 1, nbuf_s, nbuf_d)
                    _fire(nbuf_s, 0, (p0 + IB) % 2)
            else:
                _load(b + 1, nbuf_s, nbuf_d)
                _fire(nbuf_s, 0, (p0 + IB) % 2)
            _scat(sbuf, dbuf, IB - 1, (p0 + IB - 1) % 2)

        def _agg_pair(i, carry):
            _half_batch(i, 2 * i, srcbA, dstbA, srcbB, dstbB, 0, False)
            _half_batch(i, 2 * i + 1, srcbB, dstbB, srcbA, dstbA, 1, True)
            return carry
        lax.fori_loop(0, NBP, _agg_pair, 0)
        _wait_scatter(0)
        _wait_scatter(1)
        plsc.subcore_barrier()

        last = layer == NLAYERS - 1

        def _post_chunk(i, carry):
            base = row_base + i * RCHUNK
            pltpu.sync_copy(sdb.at[c].at[pl.ds(base, RCHUNK)], sb2)
            pltpu.sync_copy(acc.at[pl.ds(base, RCHUNK)], ab2)
            pltpu.sync_copy(out.at[pl.ds(base, RCHUNK), pl.ds(col, H)], ob2)

            def _rows(r0, carry2):
                for k in range(4):
                    r = r0 * 4 + k
                    for h in range(H // L):
                        sl = pl.ds(h * L, L)
                        sd = sb2[r, sl]
                        t = ab2[r, sl] * sd
                        o = ob2[r, sl] + t
                        if last:
                            ob2[r, sl] = o * 0.25
                        else:
                            ob2[r, sl] = o
                            yb2[r, sl] = t * sd
                return carry2
            lax.fori_loop(0, RCHUNK // 4, _rows, 0)
            pltpu.sync_copy(zb2, acc.at[pl.ds(base, RCHUNK)])
            pltpu.sync_copy(ob2, out.at[pl.ds(base, RCHUNK), pl.ds(col, H)])
            if not last:
                pltpu.sync_copy(yb2, ynext.at[c].at[pl.ds(base, RCHUNK)])
            return carry
        lax.fori_loop(0, NRC, _post_chunk, 0)
        plsc.subcore_barrier()


@jax.jit
def _run(embp, srcp, dstp):
    mesh = plsc.VectorSubcoreMesh(core_axis_name="c", subcore_axis_name="s")
    f = pl.kernel(
        _body,
        out_type=[
            jax.ShapeDtypeStruct((NPAD, D), jnp.float32),
            jax.ShapeDtypeStruct((NC, NPAD, H), jnp.float32),
            jax.ShapeDtypeStruct((NC, NPAD, H), jnp.float32),
            jax.ShapeDtypeStruct((NC, NPAD, H), jnp.float32),
        ],
        mesh=mesh,
        compiler_params=pltpu.CompilerParams(use_tc_tiling_on_sc=False),
        scratch_types=[
            pltpu.VMEM_SHARED((NPAD, H), jnp.float32),
            pltpu.VMEM_SHARED((NPAD,), jnp.float32),
            pltpu.VMEM((IB, CHUNK), jnp.int32),
            pltpu.VMEM((IB, CHUNK), jnp.int32),
            pltpu.VMEM((IB, CHUNK), jnp.int32),
            pltpu.VMEM((IB, CHUNK), jnp.int32),
            pltpu.VMEM((CHUNK, H), jnp.float32),
            pltpu.VMEM((CHUNK, H), jnp.float32),
            pltpu.VMEM((RCHUNK, H), jnp.float32),
            pltpu.VMEM((RCHUNK, H), jnp.float32),
            pltpu.VMEM((RCHUNK, H), jnp.float32),
            pltpu.VMEM((RCHUNK, H), jnp.float32),
            pltpu.VMEM((RCHUNK, H), jnp.float32),
            pltpu.VMEM((RCHUNK,), jnp.float32),
            pltpu.VMEM((CHUNK,), jnp.float32),
            pltpu.VMEM((RCHUNK,), jnp.float32),
            pltpu.VMEM((CHUNK,), jnp.int32),
            pltpu.SemaphoreType.DMA,
            pltpu.SemaphoreType.DMA,
            pltpu.SemaphoreType.DMA,
            pltpu.SemaphoreType.DMA,
        ],
    )
    return f(embp, srcp, dstp)


def kernel(emb, edge_index):
    src = edge_index[0]
    dst = edge_index[1]
    pad = jnp.full((EPAD - E,), DUMMY, dtype=jnp.int32)
    srcp = jnp.concatenate([src, pad]).reshape(NCHUNKS, CHUNK)
    dstp = jnp.concatenate([dst, pad]).reshape(NCHUNKS, CHUNK)
    embp = jnp.pad(emb, ((0, NPAD - N), (0, 0)))
    out, _, _, _ = _run(embp, srcp, dstp)
    return out[:N]
